# baseline (speedup 1.0000x reference)
import functools

import jax
import jax.numpy as jnp
from jax import lax
from jax.experimental import pallas as pl
from jax.experimental.pallas import tpu as pltpu
from jax.experimental.pallas import tpu_sc as plsc

DIM = 128
BATCH = 16384

_info = plsc.get_sparse_core_info()
NC, NS, L = _info.num_cores, _info.num_subcores, _info.num_lanes
NW = NC * NS
BPW = BATCH // NW
NGC = BPW // 128
NSLOT = 6

_sc_mesh = plsc.VectorSubcoreMesh(core_axis_name="c", subcore_axis_name="s")


@functools.partial(
    pl.kernel,
    mesh=_sc_mesh,
    out_type=[
        jax.ShapeDtypeStruct((BATCH, DIM), jnp.float32),
        jax.ShapeDtypeStruct((BATCH, DIM), jnp.float32),
        jax.ShapeDtypeStruct((BATCH, 16), jnp.float32),
    ],
    scratch_types=[
        pltpu.VMEM((NGC, 128), jnp.int32),
        pltpu.VMEM((NGC, 128), jnp.int32),
        pltpu.VMEM((NSLOT * 128, DIM), jnp.float32),
        pltpu.VMEM((8, 16), jnp.float32),
        pltpu.VMEM((128, 16), jnp.float32),
        pltpu.SemaphoreType.DMA,
        pltpu.SemaphoreType.DMA,
        pltpu.SemaphoreType.DMA,
        pltpu.SemaphoreType.DMA,
    ],
)
def _sc_gather(uidx_hbm, iidx_hbm, tab_mu, tab_mi, tab_gu, tab_gi, wg_hbm,
               out_mu, out_mi, out_gd, idx_u, idx_i, bufs, wgv, part,
               mgsem, ggsem, mwsem, gwsem):
    wid = lax.axis_index("s") * NC + lax.axis_index("c")
    base = wid * BPW

    def slot(s):
        return bufs.at[pl.ds(s * 128, 128)]

    def out_rows(out, c):
        return out.at[pl.ds(base + c * 128, 128)]

    pltpu.sync_copy(uidx_hbm.at[wid], idx_u)
    pltpu.sync_copy(iidx_hbm.at[wid], idx_i)
    pltpu.sync_copy(wg_hbm, wgv)

    mlp = [(tab_mu if t < 4 else tab_mi,
            (idx_u if t < 4 else idx_i).at[t % 4],
            out_rows(out_mu if t < 4 else out_mi, t % 4),
            4 + t % 2) for t in range(2 * NGC)]
    TM = len(mlp)
    mg = [None] * TM
    mw = [None] * TM
    pg = [None] * NGC
    qg = [None] * NGC
    pw = [None] * NGC

    def gmf_fire(c):
        s = 2 * (c % 2)
        pg[c] = pltpu.async_copy(tab_gu.at[idx_u.at[c]], slot(s), ggsem)
        qg[c] = pltpu.async_copy(tab_gi.at[idx_i.at[c]], slot(s + 1), ggsem)

    wvecs = [wgv[j, :] for j in range(DIM // 16)]

    def gmf_compute(c):
        pg[c].wait()
        qg[c].wait()
        if c >= 1:
            pw[c - 1].wait()
        s = 2 * (c % 2)
        a = slot(s)
        b = slot(s + 1)

        def body(r, _):
            acc = a[r, pl.ds(0, 16)] * b[r, pl.ds(0, 16)] * wvecs[0]
            for j in range(1, DIM // 16):
                sl = pl.ds(j * 16, 16)
                acc += a[r, sl] * b[r, sl] * wvecs[j]
            part[r, :] = acc
            return 0

        lax.fori_loop(0, 128, body, 0)
        pw[c] = pltpu.async_copy(part, out_rows(out_gd, c), gwsem)

    gmf_fire(0)
    gmf_fire(1)
    mg[0] = pltpu.async_copy(mlp[0][0].at[mlp[0][1]], slot(mlp[0][3]), mgsem)
    for t in range(TM):
        if t + 1 < TM:
            if t >= 1:
                mw[t - 1].wait()
            mg[t + 1] = pltpu.async_copy(mlp[t + 1][0].at[mlp[t + 1][1]],
                                         slot(mlp[t + 1][3]), mgsem)
        mg[t].wait()
        mw[t] = pltpu.async_copy(slot(mlp[t][3]), mlp[t][2], mwsem)
        if t % 2 == 0:
            gmf_compute(t // 2)
        elif t < TM - 1:
            c = (t - 1) // 2 + 2
            if c < NGC:
                gmf_fire(c)
    mw[TM - 2].wait()
    mw[TM - 1].wait()
    pw[NGC - 1].wait()


_TC_BLK = 2048


def _tc_body(mu_ref, mi_ref, gd_ref, w1a_ref, w1b_ref, b1_ref,
             w2_ref, b2_ref, w3_ref, b3_ref, wm_ref, bl_ref, out_ref):
    f32 = jnp.float32
    h = jnp.dot(mu_ref[...], w1a_ref[...], preferred_element_type=f32)
    h += jnp.dot(mi_ref[...], w1b_ref[...], preferred_element_type=f32)
    h = jnp.maximum(h + b1_ref[...], 0.0)
    h = jnp.maximum(jnp.dot(h, w2_ref[...], preferred_element_type=f32)
                    + b2_ref[...], 0.0)
    h = jnp.maximum(jnp.dot(h, w3_ref[...], preferred_element_type=f32)
                    + b3_ref[...], 0.0)
    out = jnp.sum(gd_ref[...], axis=1, keepdims=True)
    out += jnp.dot(h, wm_ref[...], preferred_element_type=f32)
    out_ref[...] = out + bl_ref[...]


def _fixed(shape):
    return pl.BlockSpec(shape, lambda b: (0, 0))


_tc_dense = pl.pallas_call(
    _tc_body,
    grid=(BATCH // _TC_BLK,),
    in_specs=[
        pl.BlockSpec((_TC_BLK, DIM), lambda b: (b, 0)),
        pl.BlockSpec((_TC_BLK, DIM), lambda b: (b, 0)),
        pl.BlockSpec((_TC_BLK, 16), lambda b: (b, 0)),
        _fixed((DIM, 64)),
        _fixed((DIM, 64)),
        _fixed((1, 64)),
        _fixed((64, 16)),
        _fixed((1, 16)),
        _fixed((16, 8)),
        _fixed((1, 8)),
        _fixed((8, 1)),
        _fixed((1, 1)),
    ],
    out_specs=pl.BlockSpec((_TC_BLK, 1), lambda b: (b, 0)),
    out_shape=jax.ShapeDtypeStruct((BATCH, 1), jnp.float32),
)


def kernel(x, mlp_user_w, mlp_item_w, gmf_user_w, gmf_item_w,
           W1, b1, W2, b2, W3, b3, W_last, b_last):
    u = x[:, 0].astype(jnp.int32).reshape(NW, NGC, 128)
    i = x[:, 1].astype(jnp.int32).reshape(NW, NGC, 128)
    wg = W_last[0, :DIM].reshape(8, 16)
    mu, mi, gd = _sc_gather(u, i, mlp_user_w, mlp_item_w,
                            gmf_user_w, gmf_item_w, wg)
    w1a = W1[:, :DIM].T
    w1b = W1[:, DIM:].T
    wm = W_last[0, DIM:].reshape(8, 1)
    return _tc_dense(mu, mi, gd, w1a, w1b, b1.reshape(1, 64),
                     W2.T, b2.reshape(1, 16), W3.T, b3.reshape(1, 8),
                     wm, b_last.reshape(1, 1))


# speedup vs baseline: 1.0356x; 1.0356x over previous
import functools

import jax
import jax.numpy as jnp
from jax import lax
from jax.experimental import pallas as pl
from jax.experimental.pallas import tpu as pltpu
from jax.experimental.pallas import tpu_sc as plsc

DIM = 128
BATCH = 16384

_info = plsc.get_sparse_core_info()
NC, NS, L = _info.num_cores, _info.num_subcores, _info.num_lanes
NW = NC * NS
BPW = BATCH // NW
NGC = BPW // 128
NSLOT = 6

_sc_mesh = plsc.VectorSubcoreMesh(core_axis_name="c", subcore_axis_name="s")


@functools.partial(
    pl.kernel,
    mesh=_sc_mesh,
    out_type=[jax.ShapeDtypeStruct((BATCH, DIM), jnp.float32) for _ in range(3)],
    scratch_types=[
        pltpu.VMEM((NGC, 128), jnp.int32),
        pltpu.VMEM((NGC, 128), jnp.int32),
        pltpu.VMEM((NSLOT * 128, DIM), jnp.float32),
        pltpu.SemaphoreType.DMA,
        pltpu.SemaphoreType.DMA,
        pltpu.SemaphoreType.DMA,
        pltpu.SemaphoreType.DMA,
    ],
)
def _sc_gather(uidx_hbm, iidx_hbm, tab_mu, tab_mi, tab_gu, tab_gi,
               out_mu, out_mi, out_g, idx_u, idx_i, bufs,
               mgsem, ggsem, mwsem, gwsem):
    wid = lax.axis_index("s") * NC + lax.axis_index("c")
    base = wid * BPW

    def slot(s):
        return bufs.at[pl.ds(s * 128, 128)]

    def out_rows(out, c):
        return out.at[pl.ds(base + c * 128, 128)]

    pltpu.sync_copy(uidx_hbm.at[wid], idx_u)
    pltpu.sync_copy(iidx_hbm.at[wid], idx_i)

    mlp = [(tab_mu if t < 4 else tab_mi,
            (idx_u if t < 4 else idx_i).at[t % 4],
            out_rows(out_mu if t < 4 else out_mi, t % 4),
            4 + t % 2) for t in range(2 * NGC)]
    TM = len(mlp)
    mg = [None] * TM
    mw = [None] * TM
    pg = [None] * NGC
    qg = [None] * NGC
    pw = [None] * NGC

    def gmf_fire(c):
        s = 2 * (c % 2)
        pg[c] = pltpu.async_copy(tab_gu.at[idx_u.at[c]], slot(s), ggsem)
        qg[c] = pltpu.async_copy(tab_gi.at[idx_i.at[c]], slot(s + 1), ggsem)

    def gmf_compute(c):
        pg[c].wait()
        qg[c].wait()
        s = 2 * (c % 2)
        a = slot(s)
        b = slot(s + 1)

        def body(r, _):
            for j in range(DIM // 16):
                sl = pl.ds(j * 16, 16)
                a[r, sl] = a[r, sl] * b[r, sl]
            return 0

        lax.fori_loop(0, 128, body, 0)
        pw[c] = pltpu.async_copy(a, out_rows(out_g, c), gwsem)

    gmf_fire(0)
    gmf_fire(1)
    mg[0] = pltpu.async_copy(mlp[0][0].at[mlp[0][1]], slot(mlp[0][3]), mgsem)
    for t in range(TM):
        if t + 1 < TM:
            if t >= 1:
                mw[t - 1].wait()
            mg[t + 1] = pltpu.async_copy(mlp[t + 1][0].at[mlp[t + 1][1]],
                                         slot(mlp[t + 1][3]), mgsem)
        mg[t].wait()
        mw[t] = pltpu.async_copy(slot(mlp[t][3]), mlp[t][2], mwsem)
        if t % 2 == 0:
            gmf_compute(t // 2)
        elif t < TM - 1:
            c = (t - 1) // 2 + 2
            if c < NGC:
                pw[c - 2].wait()
                gmf_fire(c)
    mw[TM - 2].wait()
    mw[TM - 1].wait()
    for c in range(NGC - 2, NGC):
        pw[c].wait()


_TC_BLK = 4096


def _tc_body(mu_ref, mi_ref, g_ref, w1a_ref, w1b_ref, b1_ref,
             w2_ref, b2_ref, w3_ref, b3_ref, wg_ref, wm_ref, bl_ref, out_ref):
    f32 = jnp.float32
    h = jnp.dot(mu_ref[...], w1a_ref[...], preferred_element_type=f32)
    h += jnp.dot(mi_ref[...], w1b_ref[...], preferred_element_type=f32)
    h = jnp.maximum(h + b1_ref[...], 0.0)
    h = jnp.maximum(jnp.dot(h, w2_ref[...], preferred_element_type=f32)
                    + b2_ref[...], 0.0)
    h = jnp.maximum(jnp.dot(h, w3_ref[...], preferred_element_type=f32)
                    + b3_ref[...], 0.0)
    out = jnp.dot(g_ref[...], wg_ref[...], preferred_element_type=f32)
    out += jnp.dot(h, wm_ref[...], preferred_element_type=f32)
    out_ref[...] = out + bl_ref[...]


def _fixed(shape):
    return pl.BlockSpec(shape, lambda b: (0, 0))


_tc_dense = pl.pallas_call(
    _tc_body,
    grid=(BATCH // _TC_BLK,),
    in_specs=[
        pl.BlockSpec((_TC_BLK, DIM), lambda b: (b, 0)),
        pl.BlockSpec((_TC_BLK, DIM), lambda b: (b, 0)),
        pl.BlockSpec((_TC_BLK, DIM), lambda b: (b, 0)),
        _fixed((DIM, 64)),
        _fixed((DIM, 64)),
        _fixed((1, 64)),
        _fixed((64, 16)),
        _fixed((1, 16)),
        _fixed((16, 8)),
        _fixed((1, 8)),
        _fixed((DIM, 1)),
        _fixed((8, 1)),
        _fixed((1, 1)),
    ],
    out_specs=pl.BlockSpec((_TC_BLK, 1), lambda b: (b, 0)),
    out_shape=jax.ShapeDtypeStruct((BATCH, 1), jnp.float32),
)


def kernel(x, mlp_user_w, mlp_item_w, gmf_user_w, gmf_item_w,
           W1, b1, W2, b2, W3, b3, W_last, b_last):
    u = x[:, 0].astype(jnp.int32).reshape(NW, NGC, 128)
    i = x[:, 1].astype(jnp.int32).reshape(NW, NGC, 128)
    mu, mi, g = _sc_gather(u, i, mlp_user_w, mlp_item_w,
                           gmf_user_w, gmf_item_w)
    w1a = W1[:, :DIM].T
    w1b = W1[:, DIM:].T
    wg = W_last[0, :DIM].reshape(DIM, 1)
    wm = W_last[0, DIM:].reshape(8, 1)
    return _tc_dense(mu, mi, g, w1a, w1b, b1.reshape(1, 64),
                     W2.T, b2.reshape(1, 16), W3.T, b3.reshape(1, 8),
                     wg, wm, b_last.reshape(1, 1))
